# baseline (speedup 1.0000x reference)
import functools

import jax
import jax.numpy as jnp
from jax import lax
from jax.experimental import pallas as pl
from jax.experimental.pallas import tpu as pltpu
from jax.experimental.pallas import tpu_sc as plsc

N = 10000
E = 320000
H = 128
G = 64
NC, NS, LANES = 2, 16, 16
NW = NC * NS


RPT = 312
ECH = 104
_sc_mesh = plsc.VectorSubcoreMesh(core_axis_name="c", subcore_axis_name="s",
                                  num_cores=NC, num_subcores=NS)


@functools.partial(
    pl.kernel,
    out_type=jax.ShapeDtypeStruct((N, H), jnp.float32),
    mesh=_sc_mesh,
    scratch_types=[
        pltpu.VMEM((ECH,), jnp.int32),
        pltpu.VMEM((ECH, H), jnp.float32),
        pltpu.VMEM((16,), jnp.int32),
        pltpu.VMEM((16, H), jnp.float32),
        pltpu.SemaphoreType.DMA,
    ],
)
def _emb_gather(emb_hbm, x_hbm, out_hbm, idx_v, rows_v, idx_t, rows_t, sem):
    c = lax.axis_index("c")
    s = lax.axis_index("s")
    wid = c * NS + s
    base = wid * RPT
    for ch in range(RPT // ECH):
        rb = base + ch * ECH
        pltpu.sync_copy(x_hbm.at[pl.ds(rb, ECH)], idx_v)
        pltpu.async_copy(emb_hbm.at[idx_v], rows_v, sem).wait()
        pltpu.sync_copy(rows_v, out_hbm.at[pl.ds(rb, ECH)])

    @pl.when(wid == NW - 1)
    def _tail():
        rb = NW * RPT
        pltpu.sync_copy(x_hbm.at[pl.ds(rb, 16)], idx_t)
        pltpu.async_copy(emb_hbm.at[idx_t], rows_t, sem).wait()
        pltpu.sync_copy(rows_t, out_hbm.at[pl.ds(rb, 16)])


def _vreg_gather(vec, idx):
    return lax.gather(
        vec, idx[:, None],
        dimension_numbers=lax.GatherDimensionNumbers(
            offset_dims=(), collapsed_slice_dims=(0,), start_index_map=(0,)),
        slice_sizes=(1,),
        mode=lax.GatherScatterMode.PROMISE_IN_BOUNDS)


KE = 64
NCH = 160
EPAD = NW * NCH * KE
RPS = 624
NB = 4


@functools.partial(
    pl.kernel,
    out_type=jax.ShapeDtypeStruct((NC, N, H), jnp.float32),
    mesh=_sc_mesh,
    scratch_types=[
        [pltpu.VMEM((2, KE), jnp.int32) for _ in range(NB)],
        [pltpu.VMEM((KE,), jnp.int32) for _ in range(NB)],
        [pltpu.VMEM((KE,), jnp.int32) for _ in range(NB)],
        [pltpu.VMEM((KE, H), jnp.float32) for _ in range(NB)],
        pltpu.VMEM_SHARED((N, H), jnp.float32),
        [pltpu.SemaphoreType.DMA for _ in range(NB)],
        [pltpu.SemaphoreType.DMA for _ in range(NB)],
        [pltpu.SemaphoreType.DMA for _ in range(NB)],
    ],
)
def _edge_pass(m_hbm, pk_hbm, part_hbm,
               pbuf, sbuf, dbuf, rows, agg_sh, gsem, ssem, psem):
    cc = lax.axis_index("c")
    s = lax.axis_index("s")
    wid = cc * NS + s

    def _z(i, _):
        for j in range(H // LANES):
            rows[0][i, pl.ds(j * LANES, LANES)] = jnp.zeros((LANES,),
                                                            jnp.float32)
        return 0
    lax.fori_loop(0, KE, _z, 0)
    for kk in range(RPS // KE):
        pltpu.sync_copy(rows[0], agg_sh.at[pl.ds(s * RPS + kk * KE, KE)])
    ztail = RPS - (RPS // KE) * KE
    pltpu.sync_copy(rows[0].at[pl.ds(0, ztail)],
                    agg_sh.at[pl.ds(s * RPS + (RPS // KE) * KE, ztail)])

    @pl.when(s == NS - 1)
    def _ztail():
        pltpu.sync_copy(rows[0].at[pl.ds(0, N - NS * RPS)],
                        agg_sh.at[pl.ds(NS * RPS, N - NS * RPS)])
    plsc.subcore_barrier()

    def _unpack(u):
        for gI in range(KE // LANES):
            sl = pl.ds(gI * LANES, LANES)
            p = pbuf[u][0, sl]
            sbuf[u][sl] = p & jnp.int32(0xFFFF)
            dbuf[u][sl] = lax.shift_right_logical(p, jnp.int32(16))

    def _scale(buf, u):
        def _grp(gI, _):
            we16 = lax.bitcast_convert_type(
                pbuf[u][1, pl.ds(gI * LANES, LANES)], jnp.float32)
            for j in range(LANES):
                wv = _vreg_gather(we16, jnp.full((LANES,), j, jnp.int32))
                e = gI * LANES + j
                for k in range(H // LANES):
                    sl = pl.ds(k * LANES, LANES)
                    buf[e, sl] = buf[e, sl] * wv
            return 0
        lax.fori_loop(0, KE // LANES, _grp, 0)

    pltpu.sync_copy(pk_hbm.at[wid, 0], pbuf[0])
    pltpu.sync_copy(pk_hbm.at[wid, 1], pbuf[1])
    pltpu.async_copy(pk_hbm.at[wid, 2], pbuf[2], psem[2])
    _unpack(0)
    _unpack(1)
    pltpu.async_copy(m_hbm.at[sbuf[0]], rows[0], gsem[0])
    pltpu.async_copy(m_hbm.at[sbuf[1]], rows[1], gsem[1])

    def _outer(oc, _):
        for u in range(NB):
            ci = oc * NB + u
            un = (u + 2) % NB

            @pl.when(ci + 3 < NCH)
            def _meta():
                pltpu.async_copy(pk_hbm.at[wid, ci + 3], pbuf[(u + 3) % NB],
                                 psem[(u + 3) % NB])

            @pl.when(ci + 2 < NCH)
            def _pref():
                pltpu.make_async_copy(pk_hbm.at[wid, 0], pbuf[un],
                                      psem[un]).wait()

                @pl.when(ci >= 2)
                def _drain():
                    pltpu.make_async_copy(
                        rows[un], agg_sh.at[dbuf[un]], ssem[un]).wait()
                _unpack(un)
                pltpu.async_copy(m_hbm.at[sbuf[un]], rows[un], gsem[un])

            pltpu.make_async_copy(m_hbm.at[sbuf[u]], rows[u],
                                  gsem[u]).wait()
            _scale(rows[u], u)
            pltpu.async_copy(rows[u], agg_sh.at[dbuf[u]], ssem[u], add=True)
        return 0
    lax.fori_loop(0, NCH // NB, _outer, 0)

    for u in range(NB):
        pltpu.make_async_copy(rows[u], agg_sh.at[dbuf[u]], ssem[u]).wait()

    plsc.subcore_barrier()
    pltpu.sync_copy(agg_sh.at[pl.ds(s * RPS, RPS)],
                    part_hbm.at[cc, pl.ds(s * RPS, RPS)])

    @pl.when(s == NS - 1)
    def _otail():
        pltpu.sync_copy(agg_sh.at[pl.ds(NS * RPS, N - NS * RPS)],
                        part_hbm.at[cc, pl.ds(NS * RPS, N - NS * RPS)])


BM = 400


def _mm_body(h_ref, w_ref, o_ref):
    o_ref[:] = jnp.dot(h_ref[:], w_ref[:], preferred_element_type=jnp.float32)


def _matmul(h, w):
    return pl.pallas_call(
        _mm_body,
        grid=(N // BM,),
        in_specs=[
            pl.BlockSpec((BM, H), lambda i: (i, 0)),
            pl.BlockSpec((H, H), lambda i: (0, 0)),
        ],
        out_specs=pl.BlockSpec((BM, H), lambda i: (i, 0)),
        out_shape=jax.ShapeDtypeStruct((N, H), jnp.float32),
    )(h, w)


def _post(p0, p1, b, g, beta):
    h = p0 + p1 + b
    mu = jnp.mean(h, axis=-1, keepdims=True)
    var = jnp.mean((h - mu) * (h - mu), axis=-1, keepdims=True)
    hn = (h - mu) * lax.rsqrt(var + 1e-5) * g + beta
    return jnp.maximum(hn, 0.0)


def _fuse_body(part_ref, b_ref, g_ref, beta_ref, w_ref, o_ref):
    h = _post(part_ref[0], part_ref[1], b_ref[:], g_ref[:], beta_ref[:])
    o_ref[:] = jnp.dot(h, w_ref[:], preferred_element_type=jnp.float32)


def _fuse(part, b, g, beta, w):
    return pl.pallas_call(
        _fuse_body,
        grid=(N // BM,),
        in_specs=[
            pl.BlockSpec((NC, BM, H), lambda i: (0, i, 0)),
            pl.BlockSpec((1, H), lambda i: (0, 0)),
            pl.BlockSpec((1, H), lambda i: (0, 0)),
            pl.BlockSpec((1, H), lambda i: (0, 0)),
            pl.BlockSpec((H, H), lambda i: (0, 0)),
        ],
        out_specs=pl.BlockSpec((BM, H), lambda i: (i, 0)),
        out_shape=jax.ShapeDtypeStruct((N, H), jnp.float32),
    )(part, b, g, beta, w)


BP = 400
NBP = N // BP


def _pool_body(part_ref, b_ref, g_ref, beta_ref, batch_ref,
               w1_ref, b1_ref, w2_ref, b2_ref, o_ref, acc, mx):
    i = pl.program_id(0)

    @pl.when(i == 0)
    def _init():
        acc[:] = jnp.zeros((G, 2 * H), jnp.float32)
        mx[:] = jnp.full((G, H), -jnp.inf, jnp.float32)

    h = _post(part_ref[0], part_ref[1], b_ref[:], g_ref[:], beta_ref[:])
    bb = batch_ref[:]
    gid = lax.broadcasted_iota(jnp.int32, (BP, G), 1).astype(jnp.float32)
    oh = (bb == gid).astype(jnp.float32)
    haug = jnp.concatenate([h, jnp.ones((BP, H), jnp.float32)], axis=1)
    acc[:] += lax.dot_general(oh, haug, (((0,), (0,)), ((), ())),
                              preferred_element_type=jnp.float32)
    for g in range(G):
        contrib = jnp.max(jnp.where(bb == g, h, -jnp.inf), axis=0,
                          keepdims=True)
        mx[g:g + 1, :] = jnp.maximum(mx[g:g + 1, :], contrib)

    @pl.when(i == NBP - 1)
    def _head():
        sumx = acc[:, :H]
        cnt = acc[:, H:]
        mean = sumx / jnp.maximum(cnt, 1.0)
        z1 = (jnp.dot(mean, w1_ref[:H, :], preferred_element_type=jnp.float32)
              + jnp.dot(mx[:], w1_ref[H:, :], preferred_element_type=jnp.float32)
              + b1_ref[:])
        z1 = jnp.maximum(z1, 0.0)
        o_ref[:] = (jnp.dot(z1, w2_ref[:], preferred_element_type=jnp.float32)
                    + b2_ref[:])


def _pool_head(part, b, g, beta, batchf, w1, b1, w2, b2):
    return pl.pallas_call(
        _pool_body,
        grid=(NBP,),
        in_specs=[
            pl.BlockSpec((NC, BP, H), lambda i: (0, i, 0)),
            pl.BlockSpec((1, H), lambda i: (0, 0)),
            pl.BlockSpec((1, H), lambda i: (0, 0)),
            pl.BlockSpec((1, H), lambda i: (0, 0)),
            pl.BlockSpec((BP, 1), lambda i: (i, 0)),
            pl.BlockSpec((2 * H, H), lambda i: (0, 0)),
            pl.BlockSpec((1, H), lambda i: (0, 0)),
            pl.BlockSpec((H, 1), lambda i: (0, 0)),
            pl.BlockSpec((1, 1), lambda i: (0, 0)),
        ],
        out_specs=pl.BlockSpec((G, 1), lambda i: (0, 0)),
        out_shape=jax.ShapeDtypeStruct((G, 1), jnp.float32),
        scratch_shapes=[
            pltpu.VMEM((G, 2 * H), jnp.float32),
            pltpu.VMEM((G, H), jnp.float32),
        ],
    )(part, b, g, beta, batchf, w1, b1, w2, b2)


def kernel(x, edge_index, edge_weight, batch, emb, conv_W, conv_b,
           ln_g, ln_b, W1, b1, W2, b2):
    x = x.astype(jnp.int32)
    pad = EPAD - E
    packed = (edge_index[0].astype(jnp.int32)
              + edge_index[1].astype(jnp.int32) * 65536)
    packed = jnp.concatenate([packed, jnp.zeros((pad,), jnp.int32)])
    wbits = lax.bitcast_convert_type(
        jnp.concatenate([edge_weight, jnp.zeros((pad,), jnp.float32)]),
        jnp.int32)
    meta = jnp.stack([packed.reshape(NW, NCH, KE),
                      wbits.reshape(NW, NCH, KE)], axis=2)
    batchf = batch.astype(jnp.float32).reshape(N, 1)

    h = _emb_gather(emb, x)
    m = _matmul(h, conv_W[0])
    for i in range(3):
        part = _edge_pass(m, meta)
        b_i = conv_b[i].reshape(1, H)
        g_i = ln_g[i].reshape(1, H)
        beta_i = ln_b[i].reshape(1, H)
        if i < 2:
            m = _fuse(part, b_i, g_i, beta_i, conv_W[i + 1])
        else:
            out = _pool_head(part, b_i, g_i, beta_i, batchf,
                             W1, b1.reshape(1, H), W2, b2.reshape(1, 1))
    return out[:, 0]


# speedup vs baseline: 2.9302x; 2.9302x over previous
import functools

import jax
import jax.numpy as jnp
from jax import lax
from jax.experimental import pallas as pl
from jax.experimental.pallas import tpu as pltpu
from jax.experimental.pallas import tpu_sc as plsc

N = 10000
E = 320000
H = 128
G = 64
NC, NS, LANES = 2, 16, 16
NW = NC * NS


RPT = 312
ECH = 104
_sc_mesh = plsc.VectorSubcoreMesh(core_axis_name="c", subcore_axis_name="s",
                                  num_cores=NC, num_subcores=NS)


@functools.partial(
    pl.kernel,
    out_type=jax.ShapeDtypeStruct((N, H), jnp.float32),
    mesh=_sc_mesh,
    scratch_types=[
        pltpu.VMEM((ECH,), jnp.int32),
        pltpu.VMEM((ECH, H), jnp.float32),
        pltpu.VMEM((16,), jnp.int32),
        pltpu.VMEM((16, H), jnp.float32),
        pltpu.SemaphoreType.DMA,
    ],
)
def _emb_gather(emb_hbm, x_hbm, out_hbm, idx_v, rows_v, idx_t, rows_t, sem):
    c = lax.axis_index("c")
    s = lax.axis_index("s")
    wid = c * NS + s
    base = wid * RPT
    for ch in range(RPT // ECH):
        rb = base + ch * ECH
        pltpu.sync_copy(x_hbm.at[pl.ds(rb, ECH)], idx_v)
        pltpu.async_copy(emb_hbm.at[idx_v], rows_v, sem).wait()
        pltpu.sync_copy(rows_v, out_hbm.at[pl.ds(rb, ECH)])

    @pl.when(wid == NW - 1)
    def _tail():
        rb = NW * RPT
        pltpu.sync_copy(x_hbm.at[pl.ds(rb, 16)], idx_t)
        pltpu.async_copy(emb_hbm.at[idx_t], rows_t, sem).wait()
        pltpu.sync_copy(rows_t, out_hbm.at[pl.ds(rb, 16)])


def _vreg_gather(vec, idx):
    return lax.gather(
        vec, idx[:, None],
        dimension_numbers=lax.GatherDimensionNumbers(
            offset_dims=(), collapsed_slice_dims=(0,), start_index_map=(0,)),
        slice_sizes=(1,),
        mode=lax.GatherScatterMode.PROMISE_IN_BOUNDS)


KE = 80
NCH = 128
EPAD = NW * NCH * KE
RPS = 624
NB = 4


@functools.partial(
    pl.kernel,
    out_type=jax.ShapeDtypeStruct((NC, N, H), jnp.float32),
    mesh=_sc_mesh,
    scratch_types=[
        [pltpu.VMEM((2, KE), jnp.int32) for _ in range(NB)],
        [pltpu.VMEM((KE,), jnp.int32) for _ in range(NB)],
        [pltpu.VMEM((KE,), jnp.int32) for _ in range(NB)],
        [pltpu.VMEM((KE, H), jnp.float32) for _ in range(NB)],
        pltpu.VMEM_SHARED((N, H), jnp.float32),
        [pltpu.SemaphoreType.DMA for _ in range(NB)],
        [pltpu.SemaphoreType.DMA for _ in range(NB)],
        [pltpu.SemaphoreType.DMA for _ in range(NB)],
    ],
)
def _edge_pass(m_hbm, pk_hbm, part_hbm,
               pbuf, sbuf, dbuf, rows, agg_sh, gsem, ssem, psem):
    cc = lax.axis_index("c")
    s = lax.axis_index("s")
    wid = cc * NS + s

    def _z(i, _):
        for j in range(H // LANES):
            rows[0][i, pl.ds(j * LANES, LANES)] = jnp.zeros((LANES,),
                                                            jnp.float32)
        return 0
    lax.fori_loop(0, KE, _z, 0)
    for kk in range(RPS // KE):
        pltpu.sync_copy(rows[0], agg_sh.at[pl.ds(s * RPS + kk * KE, KE)])
    ztail = RPS - (RPS // KE) * KE
    pltpu.sync_copy(rows[0].at[pl.ds(0, ztail)],
                    agg_sh.at[pl.ds(s * RPS + (RPS // KE) * KE, ztail)])

    @pl.when(s == NS - 1)
    def _ztail():
        pltpu.sync_copy(rows[0].at[pl.ds(0, N - NS * RPS)],
                        agg_sh.at[pl.ds(NS * RPS, N - NS * RPS)])
    plsc.subcore_barrier()

    def _unpack(u):
        for gI in range(KE // LANES):
            sl = pl.ds(gI * LANES, LANES)
            p = pbuf[u][0, sl]
            sbuf[u][sl] = p & jnp.int32(0xFFFF)
            dbuf[u][sl] = lax.shift_right_logical(p, jnp.int32(16))

    def _scale(buf, u):
        def _grp(gI, _):
            we16 = lax.bitcast_convert_type(
                pbuf[u][1, pl.ds(gI * LANES, LANES)], jnp.float32)
            for j in range(LANES):
                wv = _vreg_gather(we16, jnp.full((LANES,), j, jnp.int32))
                e = gI * LANES + j
                for k in range(H // LANES):
                    sl = pl.ds(k * LANES, LANES)
                    buf[e, sl] = buf[e, sl] * wv
            return 0
        lax.fori_loop(0, KE // LANES, _grp, 0)

    pltpu.sync_copy(pk_hbm.at[wid, 0], pbuf[0])
    pltpu.sync_copy(pk_hbm.at[wid, 1], pbuf[1])
    pltpu.async_copy(pk_hbm.at[wid, 2], pbuf[2], psem[2])
    _unpack(0)
    _unpack(1)
    pltpu.async_copy(m_hbm.at[sbuf[0]], rows[0], gsem[0])
    pltpu.async_copy(m_hbm.at[sbuf[1]], rows[1], gsem[1])

    def _outer(oc, _):
        for u in range(NB):
            ci = oc * NB + u
            un = (u + 2) % NB

            @pl.when(ci + 3 < NCH)
            def _meta():
                pltpu.async_copy(pk_hbm.at[wid, ci + 3], pbuf[(u + 3) % NB],
                                 psem[(u + 3) % NB])

            @pl.when(ci + 2 < NCH)
            def _pref():
                pltpu.make_async_copy(pk_hbm.at[wid, 0], pbuf[un],
                                      psem[un]).wait()

                @pl.when(ci >= 2)
                def _drain():
                    pltpu.make_async_copy(
                        rows[un], agg_sh.at[dbuf[un]], ssem[un]).wait()
                _unpack(un)
                pltpu.async_copy(m_hbm.at[sbuf[un]], rows[un], gsem[un])

            pltpu.make_async_copy(m_hbm.at[sbuf[u]], rows[u],
                                  gsem[u]).wait()
            _scale(rows[u], u)
            pltpu.async_copy(rows[u], agg_sh.at[dbuf[u]], ssem[u], add=True)
        return 0
    lax.fori_loop(0, NCH // NB, _outer, 0)

    for u in range(NB):
        pltpu.make_async_copy(rows[u], agg_sh.at[dbuf[u]], ssem[u]).wait()

    plsc.subcore_barrier()
    pltpu.sync_copy(agg_sh.at[pl.ds(s * RPS, RPS)],
                    part_hbm.at[cc, pl.ds(s * RPS, RPS)])

    @pl.when(s == NS - 1)
    def _otail():
        pltpu.sync_copy(agg_sh.at[pl.ds(NS * RPS, N - NS * RPS)],
                        part_hbm.at[cc, pl.ds(NS * RPS, N - NS * RPS)])


BM = 400


def _mm_body(h_ref, w_ref, o_ref):
    o_ref[:] = jnp.dot(h_ref[:], w_ref[:], preferred_element_type=jnp.float32)


def _matmul(h, w):
    return pl.pallas_call(
        _mm_body,
        grid=(N // BM,),
        in_specs=[
            pl.BlockSpec((BM, H), lambda i: (i, 0)),
            pl.BlockSpec((H, H), lambda i: (0, 0)),
        ],
        out_specs=pl.BlockSpec((BM, H), lambda i: (i, 0)),
        out_shape=jax.ShapeDtypeStruct((N, H), jnp.float32),
    )(h, w)


def _post(p0, p1, b, g, beta):
    h = p0 + p1 + b
    mu = jnp.mean(h, axis=-1, keepdims=True)
    var = jnp.mean((h - mu) * (h - mu), axis=-1, keepdims=True)
    hn = (h - mu) * lax.rsqrt(var + 1e-5) * g + beta
    return jnp.maximum(hn, 0.0)


def _fuse_body(part_ref, b_ref, g_ref, beta_ref, w_ref, o_ref):
    h = _post(part_ref[0], part_ref[1], b_ref[:], g_ref[:], beta_ref[:])
    o_ref[:] = jnp.dot(h, w_ref[:], preferred_element_type=jnp.float32)


def _fuse(part, b, g, beta, w):
    return pl.pallas_call(
        _fuse_body,
        grid=(N // BM,),
        in_specs=[
            pl.BlockSpec((NC, BM, H), lambda i: (0, i, 0)),
            pl.BlockSpec((1, H), lambda i: (0, 0)),
            pl.BlockSpec((1, H), lambda i: (0, 0)),
            pl.BlockSpec((1, H), lambda i: (0, 0)),
            pl.BlockSpec((H, H), lambda i: (0, 0)),
        ],
        out_specs=pl.BlockSpec((BM, H), lambda i: (i, 0)),
        out_shape=jax.ShapeDtypeStruct((N, H), jnp.float32),
    )(part, b, g, beta, w)


BP = 400
NBP = N // BP


def _pool_body(part_ref, b_ref, g_ref, beta_ref, batch_ref,
               w1_ref, b1_ref, w2_ref, b2_ref, o_ref, acc, mx):
    i = pl.program_id(0)

    @pl.when(i == 0)
    def _init():
        acc[:] = jnp.zeros((G, 2 * H), jnp.float32)
        mx[:] = jnp.full((G, H), -jnp.inf, jnp.float32)

    h = _post(part_ref[0], part_ref[1], b_ref[:], g_ref[:], beta_ref[:])
    bb = batch_ref[:]
    gid = lax.broadcasted_iota(jnp.int32, (BP, G), 1).astype(jnp.float32)
    oh = (bb == gid).astype(jnp.float32)
    haug = jnp.concatenate([h, jnp.ones((BP, H), jnp.float32)], axis=1)
    acc[:] += lax.dot_general(oh, haug, (((0,), (0,)), ((), ())),
                              preferred_element_type=jnp.float32)
    for g in range(G):
        contrib = jnp.max(jnp.where(bb == g, h, -jnp.inf), axis=0,
                          keepdims=True)
        mx[g:g + 1, :] = jnp.maximum(mx[g:g + 1, :], contrib)

    @pl.when(i == NBP - 1)
    def _head():
        sumx = acc[:, :H]
        cnt = acc[:, H:]
        mean = sumx / jnp.maximum(cnt, 1.0)
        z1 = (jnp.dot(mean, w1_ref[:H, :], preferred_element_type=jnp.float32)
              + jnp.dot(mx[:], w1_ref[H:, :], preferred_element_type=jnp.float32)
              + b1_ref[:])
        z1 = jnp.maximum(z1, 0.0)
        o_ref[:] = (jnp.dot(z1, w2_ref[:], preferred_element_type=jnp.float32)
                    + b2_ref[:])


def _pool_head(part, b, g, beta, batchf, w1, b1, w2, b2):
    return pl.pallas_call(
        _pool_body,
        grid=(NBP,),
        in_specs=[
            pl.BlockSpec((NC, BP, H), lambda i: (0, i, 0)),
            pl.BlockSpec((1, H), lambda i: (0, 0)),
            pl.BlockSpec((1, H), lambda i: (0, 0)),
            pl.BlockSpec((1, H), lambda i: (0, 0)),
            pl.BlockSpec((BP, 1), lambda i: (i, 0)),
            pl.BlockSpec((2 * H, H), lambda i: (0, 0)),
            pl.BlockSpec((1, H), lambda i: (0, 0)),
            pl.BlockSpec((H, 1), lambda i: (0, 0)),
            pl.BlockSpec((1, 1), lambda i: (0, 0)),
        ],
        out_specs=pl.BlockSpec((G, 1), lambda i: (0, 0)),
        out_shape=jax.ShapeDtypeStruct((G, 1), jnp.float32),
        scratch_shapes=[
            pltpu.VMEM((G, 2 * H), jnp.float32),
            pltpu.VMEM((G, H), jnp.float32),
        ],
    )(part, b, g, beta, batchf, w1, b1, w2, b2)


def kernel(x, edge_index, edge_weight, batch, emb, conv_W, conv_b,
           ln_g, ln_b, W1, b1, W2, b2):
    x = x.astype(jnp.int32)
    pad = EPAD - E
    spread = jnp.arange(pad, dtype=jnp.int32) % N
    packed = (edge_index[0].astype(jnp.int32)
              + edge_index[1].astype(jnp.int32) * 65536)
    packed = jnp.concatenate([packed, spread + spread * 65536])
    wbits = lax.bitcast_convert_type(
        jnp.concatenate([edge_weight, jnp.zeros((pad,), jnp.float32)]),
        jnp.int32)
    meta = jnp.stack([packed.reshape(NW, NCH, KE),
                      wbits.reshape(NW, NCH, KE)], axis=2)
    batchf = batch.astype(jnp.float32).reshape(N, 1)

    h = _emb_gather(emb, x)
    m = _matmul(h, conv_W[0])
    for i in range(3):
        part = _edge_pass(m, meta)
        b_i = conv_b[i].reshape(1, H)
        g_i = ln_g[i].reshape(1, H)
        beta_i = ln_b[i].reshape(1, H)
        if i < 2:
            m = _fuse(part, b_i, g_i, beta_i, conv_W[i + 1])
        else:
            out = _pool_head(part, b_i, g_i, beta_i, batchf,
                             W1, b1.reshape(1, H), W2, b2.reshape(1, 1))
    return out[:, 0]


# speedup vs baseline: 3.3582x; 1.1461x over previous
import functools

import jax
import jax.numpy as jnp
from jax import lax
from jax.experimental import pallas as pl
from jax.experimental.pallas import tpu as pltpu
from jax.experimental.pallas import tpu_sc as plsc

N = 10000
E = 320000
H = 128
G = 64
NC, NS, LANES = 2, 16, 16
NW = NC * NS


RPT = 312
ECH = 104
_sc_mesh = plsc.VectorSubcoreMesh(core_axis_name="c", subcore_axis_name="s",
                                  num_cores=NC, num_subcores=NS)


@functools.partial(
    pl.kernel,
    out_type=jax.ShapeDtypeStruct((N, H), jnp.float32),
    mesh=_sc_mesh,
    scratch_types=[
        pltpu.VMEM((ECH,), jnp.int32),
        pltpu.VMEM((ECH, H), jnp.float32),
        pltpu.VMEM((16,), jnp.int32),
        pltpu.VMEM((16, H), jnp.float32),
        pltpu.SemaphoreType.DMA,
    ],
)
def _emb_gather(emb_hbm, x_hbm, out_hbm, idx_v, rows_v, idx_t, rows_t, sem):
    c = lax.axis_index("c")
    s = lax.axis_index("s")
    wid = c * NS + s
    base = wid * RPT
    for ch in range(RPT // ECH):
        rb = base + ch * ECH
        pltpu.sync_copy(x_hbm.at[pl.ds(rb, ECH)], idx_v)
        pltpu.async_copy(emb_hbm.at[idx_v], rows_v, sem).wait()
        pltpu.sync_copy(rows_v, out_hbm.at[pl.ds(rb, ECH)])

    @pl.when(wid == NW - 1)
    def _tail():
        rb = NW * RPT
        pltpu.sync_copy(x_hbm.at[pl.ds(rb, 16)], idx_t)
        pltpu.async_copy(emb_hbm.at[idx_t], rows_t, sem).wait()
        pltpu.sync_copy(rows_t, out_hbm.at[pl.ds(rb, 16)])


def _vreg_gather(vec, idx):
    return lax.gather(
        vec, idx[:, None],
        dimension_numbers=lax.GatherDimensionNumbers(
            offset_dims=(), collapsed_slice_dims=(0,), start_index_map=(0,)),
        slice_sizes=(1,),
        mode=lax.GatherScatterMode.PROMISE_IN_BOUNDS)


KE = 80
NCH = 128
EPAD = NW * NCH * KE
RPS = 624
NB = 4


@functools.partial(
    pl.kernel,
    out_type=jax.ShapeDtypeStruct((NC, N, H), jnp.float32),
    mesh=_sc_mesh,
    scratch_types=[
        [pltpu.VMEM((2, KE), jnp.int32) for _ in range(NB)],
        [pltpu.VMEM((KE,), jnp.int32) for _ in range(NB)],
        [pltpu.VMEM((KE,), jnp.int32) for _ in range(NB)],
        [pltpu.VMEM((KE, H), jnp.float32) for _ in range(NB)],
        pltpu.VMEM_SHARED((N, H), jnp.float32),
        [pltpu.SemaphoreType.DMA for _ in range(NB)],
        [pltpu.SemaphoreType.DMA for _ in range(NB)],
        [pltpu.SemaphoreType.DMA for _ in range(NB)],
    ],
)
def _edge_pass(m_hbm, pk_hbm, part_hbm,
               pbuf, sbuf, dbuf, rows, agg_sh, gsem, ssem, psem):
    cc = lax.axis_index("c")
    s = lax.axis_index("s")
    wid = cc * NS + s

    def _z(i, _):
        for j in range(H // LANES):
            rows[0][i, pl.ds(j * LANES, LANES)] = jnp.zeros((LANES,),
                                                            jnp.float32)
        return 0
    lax.fori_loop(0, KE, _z, 0)
    for kk in range(RPS // KE):
        pltpu.sync_copy(rows[0], agg_sh.at[pl.ds(s * RPS + kk * KE, KE)])
    ztail = RPS - (RPS // KE) * KE
    pltpu.sync_copy(rows[0].at[pl.ds(0, ztail)],
                    agg_sh.at[pl.ds(s * RPS + (RPS // KE) * KE, ztail)])

    @pl.when(s == NS - 1)
    def _ztail():
        pltpu.sync_copy(rows[0].at[pl.ds(0, N - NS * RPS)],
                        agg_sh.at[pl.ds(NS * RPS, N - NS * RPS)])
    plsc.subcore_barrier()

    def _unpack(u):
        for gI in range(KE // LANES):
            sl = pl.ds(gI * LANES, LANES)
            p = pbuf[u][0, sl]
            sbuf[u][sl] = p & jnp.int32(0xFFFF)
            dbuf[u][sl] = lax.shift_right_logical(p, jnp.int32(16))

    def _scale(buf, u):
        def _grp(gI, _):
            we16 = lax.bitcast_convert_type(
                pbuf[u][1, pl.ds(gI * LANES, LANES)], jnp.float32)
            for j in range(LANES):
                wv = _vreg_gather(we16, jnp.full((LANES,), j, jnp.int32))
                e = gI * LANES + j
                for k in range(H // LANES):
                    sl = pl.ds(k * LANES, LANES)
                    buf[e, sl] = buf[e, sl] * wv
            return 0
        lax.fori_loop(0, KE // LANES, _grp, 0)

    pltpu.sync_copy(pk_hbm.at[wid, 0], pbuf[0])
    pltpu.sync_copy(pk_hbm.at[wid, 1], pbuf[1])
    pltpu.async_copy(pk_hbm.at[wid, 2], pbuf[2], psem[2])
    _unpack(0)
    _unpack(1)
    pltpu.async_copy(m_hbm.at[sbuf[0]], rows[0], gsem[0])
    pltpu.async_copy(m_hbm.at[sbuf[1]], rows[1], gsem[1])

    def _outer(oc, _):
        for u in range(NB):
            ci = oc * NB + u
            un = (u + 2) % NB

            @pl.when(ci + 3 < NCH)
            def _meta():
                pltpu.async_copy(pk_hbm.at[wid, ci + 3], pbuf[(u + 3) % NB],
                                 psem[(u + 3) % NB])

            @pl.when(ci + 2 < NCH)
            def _pref():
                pltpu.make_async_copy(pk_hbm.at[wid, 0], pbuf[un],
                                      psem[un]).wait()

                @pl.when(ci >= 2)
                def _drain():
                    pltpu.make_async_copy(
                        rows[un], agg_sh.at[dbuf[un]], ssem[un]).wait()
                _unpack(un)
                pltpu.async_copy(m_hbm.at[sbuf[un]], rows[un], gsem[un])

            pltpu.make_async_copy(m_hbm.at[sbuf[u]], rows[u],
                                  gsem[u]).wait()
            _scale(rows[u], u)
            pltpu.async_copy(rows[u], agg_sh.at[dbuf[u]], ssem[u], add=True)
        return 0
    lax.fori_loop(0, NCH // NB, _outer, 0)

    for u in range(NB):
        pltpu.make_async_copy(rows[u], agg_sh.at[dbuf[u]], ssem[u]).wait()

    plsc.subcore_barrier()
    pltpu.sync_copy(agg_sh.at[pl.ds(s * RPS, RPS)],
                    part_hbm.at[cc, pl.ds(s * RPS, RPS)])

    @pl.when(s == NS - 1)
    def _otail():
        pltpu.sync_copy(agg_sh.at[pl.ds(NS * RPS, N - NS * RPS)],
                        part_hbm.at[cc, pl.ds(NS * RPS, N - NS * RPS)])


BM = 400


def _mm_body(h_ref, w_ref, o_ref):
    o_ref[:] = jnp.dot(h_ref[:], w_ref[:], preferred_element_type=jnp.float32)


def _matmul(h, w):
    return pl.pallas_call(
        _mm_body,
        grid=(N // BM,),
        in_specs=[
            pl.BlockSpec((BM, H), lambda i: (i, 0)),
            pl.BlockSpec((H, H), lambda i: (0, 0)),
        ],
        out_specs=pl.BlockSpec((BM, H), lambda i: (i, 0)),
        out_shape=jax.ShapeDtypeStruct((N, H), jnp.float32),
    )(h, w)


def _post(p0, p1, b, g, beta):
    h = p0 + p1 + b
    mu = jnp.mean(h, axis=-1, keepdims=True)
    var = jnp.mean((h - mu) * (h - mu), axis=-1, keepdims=True)
    hn = (h - mu) * lax.rsqrt(var + 1e-5) * g + beta
    return jnp.maximum(hn, 0.0)


def _fuse_body(part_ref, b_ref, g_ref, beta_ref, w_ref, o_ref):
    h = _post(part_ref[0], part_ref[1], b_ref[:], g_ref[:], beta_ref[:])
    o_ref[:] = jnp.dot(h, w_ref[:], preferred_element_type=jnp.float32)


def _fuse(part, b, g, beta, w):
    return pl.pallas_call(
        _fuse_body,
        grid=(N // BM,),
        in_specs=[
            pl.BlockSpec((NC, BM, H), lambda i: (0, i, 0)),
            pl.BlockSpec((1, H), lambda i: (0, 0)),
            pl.BlockSpec((1, H), lambda i: (0, 0)),
            pl.BlockSpec((1, H), lambda i: (0, 0)),
            pl.BlockSpec((H, H), lambda i: (0, 0)),
        ],
        out_specs=pl.BlockSpec((BM, H), lambda i: (i, 0)),
        out_shape=jax.ShapeDtypeStruct((N, H), jnp.float32),
    )(part, b, g, beta, w)


BP = 1000
NBP = N // BP


def _pool_body(part_ref, b_ref, g_ref, beta_ref, batch_ref,
               w1_ref, b1_ref, w2_ref, b2_ref, o_ref, acc, mx):
    i = pl.program_id(0)

    @pl.when(i == 0)
    def _init():
        acc[:] = jnp.zeros((G, 2 * H), jnp.float32)
        mx[:] = jnp.full((G, H), -jnp.inf, jnp.float32)

    h = _post(part_ref[0], part_ref[1], b_ref[:], g_ref[:], beta_ref[:])
    bb = batch_ref[:]
    gid = lax.broadcasted_iota(jnp.int32, (BP, G), 1).astype(jnp.float32)
    oh = (bb == gid).astype(jnp.float32)
    haug = jnp.concatenate([h, jnp.ones((BP, H), jnp.float32)], axis=1)
    acc[:] += lax.dot_general(oh, haug, (((0,), (0,)), ((), ())),
                              preferred_element_type=jnp.float32)
    bmin = batch_ref[0, 0]
    bmax = batch_ref[BP - 1, 0]
    for gc in range(G // 8):
        @pl.when(jnp.logical_and(bmax >= gc * 8, bmin <= gc * 8 + 7))
        def _chunk():
            for g in range(gc * 8, gc * 8 + 8):
                contrib = jnp.max(jnp.where(bb == g, h, -jnp.inf), axis=0,
                                  keepdims=True)
                mx[g:g + 1, :] = jnp.maximum(mx[g:g + 1, :], contrib)

    @pl.when(i == NBP - 1)
    def _head():
        sumx = acc[:, :H]
        cnt = acc[:, H:]
        mean = sumx / jnp.maximum(cnt, 1.0)
        z1 = (jnp.dot(mean, w1_ref[:H, :], preferred_element_type=jnp.float32)
              + jnp.dot(mx[:], w1_ref[H:, :], preferred_element_type=jnp.float32)
              + b1_ref[:])
        z1 = jnp.maximum(z1, 0.0)
        o_ref[:] = (jnp.dot(z1, w2_ref[:], preferred_element_type=jnp.float32)
                    + b2_ref[:])


def _pool_head(part, b, g, beta, batchf, w1, b1, w2, b2):
    return pl.pallas_call(
        _pool_body,
        grid=(NBP,),
        in_specs=[
            pl.BlockSpec((NC, BP, H), lambda i: (0, i, 0)),
            pl.BlockSpec((1, H), lambda i: (0, 0)),
            pl.BlockSpec((1, H), lambda i: (0, 0)),
            pl.BlockSpec((1, H), lambda i: (0, 0)),
            pl.BlockSpec((BP, 1), lambda i: (i, 0)),
            pl.BlockSpec((2 * H, H), lambda i: (0, 0)),
            pl.BlockSpec((1, H), lambda i: (0, 0)),
            pl.BlockSpec((H, 1), lambda i: (0, 0)),
            pl.BlockSpec((1, 1), lambda i: (0, 0)),
        ],
        out_specs=pl.BlockSpec((G, 1), lambda i: (0, 0)),
        out_shape=jax.ShapeDtypeStruct((G, 1), jnp.float32),
        scratch_shapes=[
            pltpu.VMEM((G, 2 * H), jnp.float32),
            pltpu.VMEM((G, H), jnp.float32),
        ],
    )(part, b, g, beta, batchf, w1, b1, w2, b2)


def kernel(x, edge_index, edge_weight, batch, emb, conv_W, conv_b,
           ln_g, ln_b, W1, b1, W2, b2):
    x = x.astype(jnp.int32)
    pad = EPAD - E
    spread = jnp.arange(pad, dtype=jnp.int32) % N
    packed = (edge_index[0].astype(jnp.int32)
              + edge_index[1].astype(jnp.int32) * 65536)
    packed = jnp.concatenate([packed, spread + spread * 65536])
    wbits = lax.bitcast_convert_type(
        jnp.concatenate([edge_weight, jnp.zeros((pad,), jnp.float32)]),
        jnp.int32)
    meta = jnp.stack([packed.reshape(NW, NCH, KE),
                      wbits.reshape(NW, NCH, KE)], axis=2)
    batchf = batch.astype(jnp.float32).reshape(N, 1)

    h = _emb_gather(emb, x)
    m = _matmul(h, conv_W[0])
    for i in range(3):
        part = _edge_pass(m, meta)
        b_i = conv_b[i].reshape(1, H)
        g_i = ln_g[i].reshape(1, H)
        beta_i = ln_b[i].reshape(1, H)
        if i < 2:
            m = _fuse(part, b_i, g_i, beta_i, conv_W[i + 1])
        else:
            out = _pool_head(part, b_i, g_i, beta_i, batchf,
                             W1, b1.reshape(1, H), W2, b2.reshape(1, 1))
    return out[:, 0]


# speedup vs baseline: 3.4458x; 1.0261x over previous
import functools

import jax
import jax.numpy as jnp
from jax import lax
from jax.experimental import pallas as pl
from jax.experimental.pallas import tpu as pltpu
from jax.experimental.pallas import tpu_sc as plsc

N = 10000
E = 320000
H = 128
G = 64
NC, NS, LANES = 2, 16, 16
NW = NC * NS


RPT = 312
ECH = 104
_sc_mesh = plsc.VectorSubcoreMesh(core_axis_name="c", subcore_axis_name="s",
                                  num_cores=NC, num_subcores=NS)


@functools.partial(
    pl.kernel,
    out_type=jax.ShapeDtypeStruct((N, H), jnp.float32),
    mesh=_sc_mesh,
    scratch_types=[
        pltpu.VMEM((ECH,), jnp.int32),
        pltpu.VMEM((ECH, H), jnp.float32),
        pltpu.VMEM((16,), jnp.int32),
        pltpu.VMEM((16, H), jnp.float32),
        pltpu.SemaphoreType.DMA,
    ],
)
def _emb_gather(emb_hbm, x_hbm, out_hbm, idx_v, rows_v, idx_t, rows_t, sem):
    c = lax.axis_index("c")
    s = lax.axis_index("s")
    wid = c * NS + s
    base = wid * RPT
    for ch in range(RPT // ECH):
        rb = base + ch * ECH
        pltpu.sync_copy(x_hbm.at[pl.ds(rb, ECH)], idx_v)
        pltpu.async_copy(emb_hbm.at[idx_v], rows_v, sem).wait()
        pltpu.sync_copy(rows_v, out_hbm.at[pl.ds(rb, ECH)])

    @pl.when(wid == NW - 1)
    def _tail():
        rb = NW * RPT
        pltpu.sync_copy(x_hbm.at[pl.ds(rb, 16)], idx_t)
        pltpu.async_copy(emb_hbm.at[idx_t], rows_t, sem).wait()
        pltpu.sync_copy(rows_t, out_hbm.at[pl.ds(rb, 16)])


def _vreg_gather(vec, idx):
    return lax.gather(
        vec, idx[:, None],
        dimension_numbers=lax.GatherDimensionNumbers(
            offset_dims=(), collapsed_slice_dims=(0,), start_index_map=(0,)),
        slice_sizes=(1,),
        mode=lax.GatherScatterMode.PROMISE_IN_BOUNDS)


KE = 80
NCH = 128
EPAD = NW * NCH * KE
RPS = 624
NB = 4


@functools.partial(
    pl.kernel,
    out_type=jax.ShapeDtypeStruct((NC, N, H), jnp.float32),
    mesh=_sc_mesh,
    scratch_types=[
        [pltpu.VMEM((2, KE), jnp.int32) for _ in range(NB)],
        [pltpu.VMEM((KE,), jnp.int32) for _ in range(NB)],
        [pltpu.VMEM((KE,), jnp.int32) for _ in range(NB)],
        [pltpu.VMEM((KE, H), jnp.float32) for _ in range(NB)],
        pltpu.VMEM_SHARED((N, H), jnp.float32),
        [pltpu.SemaphoreType.DMA for _ in range(NB)],
        [pltpu.SemaphoreType.DMA for _ in range(NB)],
        [pltpu.SemaphoreType.DMA for _ in range(NB)],
    ],
)
def _edge_pass(m_hbm, pk_hbm, part_hbm,
               pbuf, sbuf, dbuf, rows, agg_sh, gsem, ssem, psem):
    cc = lax.axis_index("c")
    s = lax.axis_index("s")
    wid = cc * NS + s

    def _z(i, _):
        for j in range(H // LANES):
            rows[0][i, pl.ds(j * LANES, LANES)] = jnp.zeros((LANES,),
                                                            jnp.float32)
        return 0
    lax.fori_loop(0, KE, _z, 0)
    for kk in range(RPS // KE):
        pltpu.sync_copy(rows[0], agg_sh.at[pl.ds(s * RPS + kk * KE, KE)])
    ztail = RPS - (RPS // KE) * KE
    pltpu.sync_copy(rows[0].at[pl.ds(0, ztail)],
                    agg_sh.at[pl.ds(s * RPS + (RPS // KE) * KE, ztail)])

    @pl.when(s == NS - 1)
    def _ztail():
        pltpu.sync_copy(rows[0].at[pl.ds(0, N - NS * RPS)],
                        agg_sh.at[pl.ds(NS * RPS, N - NS * RPS)])
    plsc.subcore_barrier()

    def _unpack(u):
        for gI in range(KE // LANES):
            sl = pl.ds(gI * LANES, LANES)
            p = pbuf[u][0, sl]
            sbuf[u][sl] = p & jnp.int32(0xFFFF)
            dbuf[u][sl] = lax.shift_right_logical(p, jnp.int32(16))

    def _scale(buf, u):
        def _grp(gI, _):
            we16 = lax.bitcast_convert_type(
                pbuf[u][1, pl.ds(gI * LANES, LANES)], jnp.float32)
            for j in range(LANES):
                wv = _vreg_gather(we16, jnp.full((LANES,), j, jnp.int32))
                e = gI * LANES + j
                for k in range(H // LANES):
                    sl = pl.ds(k * LANES, LANES)
                    buf[e, sl] = buf[e, sl] * wv
            return 0
        lax.fori_loop(0, KE // LANES, _grp, 0)

    pltpu.sync_copy(pk_hbm.at[wid, 0], pbuf[0])
    pltpu.sync_copy(pk_hbm.at[wid, 1], pbuf[1])
    pltpu.async_copy(pk_hbm.at[wid, 2], pbuf[2], psem[2])
    _unpack(0)
    _unpack(1)
    pltpu.async_copy(m_hbm.at[sbuf[0]], rows[0], gsem[0])
    pltpu.async_copy(m_hbm.at[sbuf[1]], rows[1], gsem[1])

    def _outer(oc, _):
        for u in range(NB):
            ci = oc * NB + u
            un = (u + 2) % NB

            @pl.when(ci + 3 < NCH)
            def _meta():
                pltpu.async_copy(pk_hbm.at[wid, ci + 3], pbuf[(u + 3) % NB],
                                 psem[(u + 3) % NB])

            @pl.when(ci + 2 < NCH)
            def _pref():
                pltpu.make_async_copy(pk_hbm.at[wid, 0], pbuf[un],
                                      psem[un]).wait()

                @pl.when(ci >= 2)
                def _drain():
                    pltpu.make_async_copy(
                        rows[un], agg_sh.at[dbuf[un]], ssem[un]).wait()
                _unpack(un)
                pltpu.async_copy(m_hbm.at[sbuf[un]], rows[un], gsem[un])

            pltpu.make_async_copy(m_hbm.at[sbuf[u]], rows[u],
                                  gsem[u]).wait()
            _scale(rows[u], u)
            pltpu.async_copy(rows[u], agg_sh.at[dbuf[u]], ssem[u], add=True)
        return 0
    lax.fori_loop(0, NCH // NB, _outer, 0)

    for u in range(NB):
        pltpu.make_async_copy(rows[u], agg_sh.at[dbuf[u]], ssem[u]).wait()

    plsc.subcore_barrier()
    pltpu.sync_copy(agg_sh.at[pl.ds(s * RPS, RPS)],
                    part_hbm.at[cc, pl.ds(s * RPS, RPS)])

    @pl.when(s == NS - 1)
    def _otail():
        pltpu.sync_copy(agg_sh.at[pl.ds(NS * RPS, N - NS * RPS)],
                        part_hbm.at[cc, pl.ds(NS * RPS, N - NS * RPS)])


BM = 400


def _mm_body(h_ref, w_ref, o_ref):
    o_ref[:] = jnp.dot(h_ref[:], w_ref[:], preferred_element_type=jnp.float32)


def _matmul(h, w):
    return pl.pallas_call(
        _mm_body,
        grid=(N // BM,),
        in_specs=[
            pl.BlockSpec((BM, H), lambda i: (i, 0)),
            pl.BlockSpec((H, H), lambda i: (0, 0)),
        ],
        out_specs=pl.BlockSpec((BM, H), lambda i: (i, 0)),
        out_shape=jax.ShapeDtypeStruct((N, H), jnp.float32),
    )(h, w)


def _post2(agg, b, g, beta):
    h = agg + b
    mu = jnp.mean(h, axis=-1, keepdims=True)
    var = jnp.mean((h - mu) * (h - mu), axis=-1, keepdims=True)
    hn = (h - mu) * lax.rsqrt(var + 1e-5) * g + beta
    return jnp.maximum(hn, 0.0)


def _post(part_ref, w_ref, b, g, beta):
    agg = jnp.dot(part_ref[0] + part_ref[1], w_ref[:],
                  preferred_element_type=jnp.float32)
    return _post2(agg, b, g, beta)


def _fuse_body(part_ref, w_ref, b_ref, g_ref, beta_ref, o_ref):
    agg = jnp.dot(part_ref[0] + part_ref[1], w_ref[:],
                  preferred_element_type=jnp.float32)
    o_ref[:] = _post2(agg, b_ref[:], g_ref[:], beta_ref[:])


def _fuse(part, w, b, g, beta):
    return pl.pallas_call(
        _fuse_body,
        grid=(N // BM,),
        in_specs=[
            pl.BlockSpec((NC, BM, H), lambda i: (0, i, 0)),
            pl.BlockSpec((H, H), lambda i: (0, 0)),
            pl.BlockSpec((1, H), lambda i: (0, 0)),
            pl.BlockSpec((1, H), lambda i: (0, 0)),
            pl.BlockSpec((1, H), lambda i: (0, 0)),
        ],
        out_specs=pl.BlockSpec((BM, H), lambda i: (i, 0)),
        out_shape=jax.ShapeDtypeStruct((N, H), jnp.float32),
    )(part, w, b, g, beta)


BP = 1000
NBP = N // BP


def _pool_body(part_ref, wc_ref, b_ref, g_ref, beta_ref, batch_ref,
               w1_ref, b1_ref, w2_ref, b2_ref, o_ref, acc, mx):
    i = pl.program_id(0)

    @pl.when(i == 0)
    def _init():
        acc[:] = jnp.zeros((G, 2 * H), jnp.float32)
        mx[:] = jnp.full((G, H), -jnp.inf, jnp.float32)

    h = _post(part_ref, wc_ref, b_ref[:], g_ref[:], beta_ref[:])
    bb = batch_ref[:]
    gid = lax.broadcasted_iota(jnp.int32, (BP, G), 1).astype(jnp.float32)
    oh = (bb == gid).astype(jnp.float32)
    haug = jnp.concatenate([h, jnp.ones((BP, H), jnp.float32)], axis=1)
    acc[:] += lax.dot_general(oh, haug, (((0,), (0,)), ((), ())),
                              preferred_element_type=jnp.float32)
    bmin = batch_ref[0, 0]
    bmax = batch_ref[BP - 1, 0]
    for gc in range(G // 8):
        @pl.when(jnp.logical_and(bmax >= gc * 8, bmin <= gc * 8 + 7))
        def _chunk():
            for g in range(gc * 8, gc * 8 + 8):
                contrib = jnp.max(jnp.where(bb == g, h, -jnp.inf), axis=0,
                                  keepdims=True)
                mx[g:g + 1, :] = jnp.maximum(mx[g:g + 1, :], contrib)

    @pl.when(i == NBP - 1)
    def _head():
        sumx = acc[:, :H]
        cnt = acc[:, H:]
        mean = sumx / jnp.maximum(cnt, 1.0)
        z1 = (jnp.dot(mean, w1_ref[:H, :], preferred_element_type=jnp.float32)
              + jnp.dot(mx[:], w1_ref[H:, :], preferred_element_type=jnp.float32)
              + b1_ref[:])
        z1 = jnp.maximum(z1, 0.0)
        o_ref[:] = (jnp.dot(z1, w2_ref[:], preferred_element_type=jnp.float32)
                    + b2_ref[:])


def _pool_head(part, wc, b, g, beta, batchf, w1, b1, w2, b2):
    return pl.pallas_call(
        _pool_body,
        grid=(NBP,),
        in_specs=[
            pl.BlockSpec((NC, BP, H), lambda i: (0, i, 0)),
            pl.BlockSpec((H, H), lambda i: (0, 0)),
            pl.BlockSpec((1, H), lambda i: (0, 0)),
            pl.BlockSpec((1, H), lambda i: (0, 0)),
            pl.BlockSpec((1, H), lambda i: (0, 0)),
            pl.BlockSpec((BP, 1), lambda i: (i, 0)),
            pl.BlockSpec((2 * H, H), lambda i: (0, 0)),
            pl.BlockSpec((1, H), lambda i: (0, 0)),
            pl.BlockSpec((H, 1), lambda i: (0, 0)),
            pl.BlockSpec((1, 1), lambda i: (0, 0)),
        ],
        out_specs=pl.BlockSpec((G, 1), lambda i: (0, 0)),
        out_shape=jax.ShapeDtypeStruct((G, 1), jnp.float32),
        scratch_shapes=[
            pltpu.VMEM((G, 2 * H), jnp.float32),
            pltpu.VMEM((G, H), jnp.float32),
        ],
    )(part, wc, b, g, beta, batchf, w1, b1, w2, b2)


def kernel(x, edge_index, edge_weight, batch, emb, conv_W, conv_b,
           ln_g, ln_b, W1, b1, W2, b2):
    x = x.astype(jnp.int32)
    pad = EPAD - E
    spread = jnp.arange(pad, dtype=jnp.int32) % N
    packed = (edge_index[0].astype(jnp.int32)
              + edge_index[1].astype(jnp.int32) * 65536)
    packed = jnp.concatenate([packed, spread + spread * 65536])
    wbits = lax.bitcast_convert_type(
        jnp.concatenate([edge_weight, jnp.zeros((pad,), jnp.float32)]),
        jnp.int32)
    meta = jnp.stack([packed.reshape(NW, NCH, KE),
                      wbits.reshape(NW, NCH, KE)], axis=2)
    batchf = batch.astype(jnp.float32).reshape(N, 1)

    h = _emb_gather(emb, x)
    for i in range(3):
        part = _edge_pass(h, meta)
        b_i = conv_b[i].reshape(1, H)
        g_i = ln_g[i].reshape(1, H)
        beta_i = ln_b[i].reshape(1, H)
        if i < 2:
            h = _fuse(part, conv_W[i], b_i, g_i, beta_i)
        else:
            out = _pool_head(part, conv_W[i], b_i, g_i, beta_i, batchf,
                             W1, b1.reshape(1, H), W2, b2.reshape(1, 1))
    return out[:, 0]
